# baseline (speedup 1.0000x reference)
import functools

import jax
import jax.numpy as jnp
from jax import lax
from jax.experimental import pallas as pl
from jax.experimental.pallas import tpu as pltpu
from jax.experimental.pallas import tpu_sc as plsc

B = 16384
D = 32
PACK = 128 // D
NC = 2
NS = 16
NW = NC * NS
BPW = B // NW
CHUNK = 256
IDXW = 128


def _sc_body(user_hbm, item_hbm, uf4_hbm, if4_hbm, out_hbm,
             uidx, iidx, urid, irid, urows, irows, sbuf, out_v,
             sem_u, sem_i):
    wid = lax.axis_index("s") * NC + lax.axis_index("c")
    base = wid * BPW

    for k in range(BPW // IDXW):
        pltpu.sync_copy(user_hbm.at[pl.ds(base + k * IDXW, IDXW)],
                        uidx.at[k])
        pltpu.sync_copy(item_hbm.at[pl.ds(base + k * IDXW, IDXW)],
                        iidx.at[k])

    for k in range(BPW // IDXW):
        for s in range(IDXW // 16):
            sl = pl.ds(s * 16, 16)
            urid[k, sl] = lax.shift_right_logical(uidx[k, sl], 2)
            irid[k, sl] = lax.shift_right_logical(iidx[k, sl], 2)

    lane = lax.broadcasted_iota(jnp.int32, (16,), 0)

    for c in range(BPW // CHUNK):
        copies = []
        for k in range(CHUNK // IDXW):
            kk = c * (CHUNK // IDXW) + k
            copies.append(pltpu.async_copy(
                uf4_hbm.at[urid.at[kk]],
                urows.at[pl.ds(k * IDXW, IDXW)], sem_u))
            copies.append(pltpu.async_copy(
                if4_hbm.at[irid.at[kk]],
                irows.at[pl.ds(k * IDXW, IDXW)], sem_i))
        for cp in copies:
            cp.wait()

        def extract(g, _):
            uv16 = uidx[c * 2 + g // 8, pl.ds((g % 8) * 16, 16)]
            iv16 = iidx[c * 2 + g // 8, pl.ds((g % 8) * 16, 16)]
            for t in range(16):
                j = g * 16 + t
                ou = (uv16[t] & 3) * D
                oi = (iv16[t] & 3) * D
                a = urows[j, pl.ds(ou, 16)] * irows[j, pl.ds(oi, 16)]
                bb = urows[j, pl.ds(ou + 16, 16)] * irows[j, pl.ds(oi + 16, 16)]
                sbuf[pl.ds((c * CHUNK + j) * 16, 16)] = a + bb
            return 0

        lax.fori_loop(0, CHUNK // 16, extract, 0)

    def group(g, _):
        base16 = g * 256 + lane * 16
        acc = plsc.load_gather(sbuf, [base16])
        for j in range(1, 16):
            acc = acc + plsc.load_gather(sbuf, [base16 + j])
        out_v[pl.ds(g * 16, 16)] = acc
        return 0

    lax.fori_loop(0, BPW // 16, group, 0)

    pltpu.sync_copy(out_v, out_hbm.at[pl.ds(base, BPW)])


@jax.jit
def kernel(user, item, user_factors, item_factors):
    uf4 = user_factors.reshape(-1, PACK * D)
    if4 = item_factors.reshape(-1, PACK * D)
    mesh = plsc.VectorSubcoreMesh(core_axis_name="c", subcore_axis_name="s")
    run = pl.kernel(
        _sc_body,
        out_type=jax.ShapeDtypeStruct((B,), jnp.float32),
        mesh=mesh,
        scratch_types=[
            pltpu.VMEM((BPW // IDXW, IDXW), jnp.int32),
            pltpu.VMEM((BPW // IDXW, IDXW), jnp.int32),
            pltpu.VMEM((BPW // IDXW, IDXW), jnp.int32),
            pltpu.VMEM((BPW // IDXW, IDXW), jnp.int32),
            pltpu.VMEM((CHUNK, PACK * D), jnp.float32),
            pltpu.VMEM((CHUNK, PACK * D), jnp.float32),
            pltpu.VMEM((BPW * 16,), jnp.float32),
            pltpu.VMEM((BPW,), jnp.float32),
            pltpu.SemaphoreType.DMA,
            pltpu.SemaphoreType.DMA,
        ],
        compiler_params=pltpu.CompilerParams(needs_layout_passes=False),
    )
    return run(user.astype(jnp.int32), item.astype(jnp.int32), uf4, if4)


# speedup vs baseline: 1.3730x; 1.3730x over previous
import functools

import jax
import jax.numpy as jnp
from jax import lax
from jax.experimental import pallas as pl
from jax.experimental.pallas import tpu as pltpu
from jax.experimental.pallas import tpu_sc as plsc

B = 16384
D = 32
NC = 2
NS = 16
NW = NC * NS
BPW = B // NW
PASS = 32
NPASS = BPW // PASS


def _sc_body(user_hbm, item_hbm, uf_hbm, if_hbm, out_hbm,
             uidx, iidx, ubufs, ibufs, sbuf, out_v, sem_u, sem_i):
    wid = lax.axis_index("s") * NC + lax.axis_index("c")
    base = wid * BPW

    pltpu.sync_copy(user_hbm.at[pl.ds(base, BPW)], uidx)
    pltpu.sync_copy(item_hbm.at[pl.ds(base, BPW)], iidx)

    lane = lax.broadcasted_iota(jnp.int32, (16,), 0)

    for p in range(NPASS):
        def fire(g, _):
            j0 = p * PASS + g * 16
            uvec = uidx[pl.ds(j0, 16)]
            ivec = iidx[pl.ds(j0, 16)]
            for t in range(16):
                ur0 = pl.multiple_of((uvec[t] >> 3) << 3, 8)
                ir0 = pl.multiple_of((ivec[t] >> 3) << 3, 8)
                dj = pl.multiple_of((g * 16 + t) * 8, 8)
                pltpu.async_copy(uf_hbm.at[pl.ds(ur0, 8)],
                                 ubufs.at[pl.ds(dj, 8)], sem_u)
                pltpu.async_copy(if_hbm.at[pl.ds(ir0, 8)],
                                 ibufs.at[pl.ds(dj, 8)], sem_i)
            return 0

        lax.fori_loop(0, PASS // 16, fire, 0)

        pltpu.make_async_copy(uf_hbm.at[pl.ds(0, PASS * 8)], ubufs,
                              sem_u).wait()
        pltpu.make_async_copy(if_hbm.at[pl.ds(0, PASS * 8)], ibufs,
                              sem_i).wait()

        def fold(g, _):
            j0 = p * PASS + g * 16
            uvec = uidx[pl.ds(j0, 16)]
            ivec = iidx[pl.ds(j0, 16)]
            for t in range(16):
                ru = (g * 16 + t) * 8 + (uvec[t] & 7)
                ri = (g * 16 + t) * 8 + (ivec[t] & 7)
                a = ubufs[ru, pl.ds(0, 16)] * ibufs[ri, pl.ds(0, 16)]
                bb = ubufs[ru, pl.ds(16, 16)] * ibufs[ri, pl.ds(16, 16)]
                sbuf[pl.ds((j0 + t) * 16, 16)] = a + bb
            return 0

        lax.fori_loop(0, PASS // 16, fold, 0)

    def group(g, _):
        base16 = g * 256 + lane * 16
        acc = plsc.load_gather(sbuf, [base16])
        for j in range(1, 16):
            acc = acc + plsc.load_gather(sbuf, [base16 + j])
        out_v[pl.ds(g * 16, 16)] = acc
        return 0

    lax.fori_loop(0, BPW // 16, group, 0)

    pltpu.sync_copy(out_v, out_hbm.at[pl.ds(base, BPW)])


@jax.jit
def kernel(user, item, user_factors, item_factors):
    mesh = plsc.VectorSubcoreMesh(core_axis_name="c", subcore_axis_name="s")
    run = pl.kernel(
        _sc_body,
        out_type=jax.ShapeDtypeStruct((B,), jnp.float32),
        mesh=mesh,
        scratch_types=[
            pltpu.VMEM((BPW,), jnp.int32),
            pltpu.VMEM((BPW,), jnp.int32),
            pltpu.VMEM((PASS * 8, D), jnp.float32),
            pltpu.VMEM((PASS * 8, D), jnp.float32),
            pltpu.VMEM((BPW * 16,), jnp.float32),
            pltpu.VMEM((BPW,), jnp.float32),
            pltpu.SemaphoreType.DMA,
            pltpu.SemaphoreType.DMA,
        ],
        compiler_params=pltpu.CompilerParams(needs_layout_passes=False),
    )
    return run(user.astype(jnp.int32), item.astype(jnp.int32),
               user_factors, item_factors)


# speedup vs baseline: 1.3936x; 1.0150x over previous
import functools

import jax
import jax.numpy as jnp
from jax import lax
from jax.experimental import pallas as pl
from jax.experimental.pallas import tpu as pltpu
from jax.experimental.pallas import tpu_sc as plsc

B = 16384
D = 32
NC = 2
NS = 16
NW = NC * NS
BPW = B // NW
PASS = 16
NPASS = BPW // PASS


def _sc_body(user_hbm, item_hbm, uf_hbm, if_hbm, out_hbm,
             uidx, iidx, ubufs_a, ibufs_a, ubufs_b, ibufs_b, sbuf, out_v,
             sem_ua, sem_ia, sem_ub, sem_ib):
    wid = lax.axis_index("s") * NC + lax.axis_index("c")
    base = wid * BPW

    pltpu.sync_copy(user_hbm.at[pl.ds(base, BPW)], uidx)
    pltpu.sync_copy(item_hbm.at[pl.ds(base, BPW)], iidx)

    lane = lax.broadcasted_iota(jnp.int32, (16,), 0)

    def fire(j0, ubufs, ibufs, sem_u, sem_i):
        uvec = uidx[pl.ds(j0, 16)]
        ivec = iidx[pl.ds(j0, 16)]
        for t in range(16):
            ur0 = pl.multiple_of((uvec[t] >> 3) << 3, 8)
            ir0 = pl.multiple_of((ivec[t] >> 3) << 3, 8)
            pltpu.async_copy(uf_hbm.at[pl.ds(ur0, 8)],
                             ubufs.at[pl.ds(t * 8, 8)], sem_u)
            pltpu.async_copy(if_hbm.at[pl.ds(ir0, 8)],
                             ibufs.at[pl.ds(t * 8, 8)], sem_i)

    def drain(ubufs, ibufs, sem_u, sem_i):
        pltpu.make_async_copy(uf_hbm.at[pl.ds(0, PASS * 8)], ubufs,
                              sem_u).wait()
        pltpu.make_async_copy(if_hbm.at[pl.ds(0, PASS * 8)], ibufs,
                              sem_i).wait()

    def fold(j0, ubufs, ibufs):
        uvec = uidx[pl.ds(j0, 16)]
        ivec = iidx[pl.ds(j0, 16)]
        for t in range(16):
            ru = t * 8 + (uvec[t] & 7)
            ri = t * 8 + (ivec[t] & 7)
            a = ubufs[ru, pl.ds(0, 16)] * ibufs[ri, pl.ds(0, 16)]
            bb = ubufs[ru, pl.ds(16, 16)] * ibufs[ri, pl.ds(16, 16)]
            sbuf[pl.ds((j0 + t) * 16, 16)] = a + bb

    def pair(q, _):
        j0 = q * (2 * PASS)
        fire(j0, ubufs_a, ibufs_a, sem_ua, sem_ia)
        fire(j0 + PASS, ubufs_b, ibufs_b, sem_ub, sem_ib)
        drain(ubufs_a, ibufs_a, sem_ua, sem_ia)
        fold(j0, ubufs_a, ibufs_a)
        drain(ubufs_b, ibufs_b, sem_ub, sem_ib)
        fold(j0 + PASS, ubufs_b, ibufs_b)
        return 0

    lax.fori_loop(0, NPASS // 2, pair, 0)

    def group(g, _):
        base16 = g * 256 + lane * 16
        acc = plsc.load_gather(sbuf, [base16])
        for j in range(1, 16):
            acc = acc + plsc.load_gather(sbuf, [base16 + j])
        out_v[pl.ds(g * 16, 16)] = acc
        return 0

    lax.fori_loop(0, BPW // 16, group, 0)

    pltpu.sync_copy(out_v, out_hbm.at[pl.ds(base, BPW)])


@jax.jit
def kernel(user, item, user_factors, item_factors):
    mesh = plsc.VectorSubcoreMesh(core_axis_name="c", subcore_axis_name="s")
    run = pl.kernel(
        _sc_body,
        out_type=jax.ShapeDtypeStruct((B,), jnp.float32),
        mesh=mesh,
        scratch_types=[
            pltpu.VMEM((BPW,), jnp.int32),
            pltpu.VMEM((BPW,), jnp.int32),
            pltpu.VMEM((PASS * 8, D), jnp.float32),
            pltpu.VMEM((PASS * 8, D), jnp.float32),
            pltpu.VMEM((PASS * 8, D), jnp.float32),
            pltpu.VMEM((PASS * 8, D), jnp.float32),
            pltpu.VMEM((BPW * 16,), jnp.float32),
            pltpu.VMEM((BPW,), jnp.float32),
            pltpu.SemaphoreType.DMA,
            pltpu.SemaphoreType.DMA,
            pltpu.SemaphoreType.DMA,
            pltpu.SemaphoreType.DMA,
        ],
        compiler_params=pltpu.CompilerParams(needs_layout_passes=False),
    )
    return run(user.astype(jnp.int32), item.astype(jnp.int32),
               user_factors, item_factors)


# speedup vs baseline: 1.4438x; 1.0360x over previous
import jax
import jax.numpy as jnp
from jax import lax
from jax.experimental import pallas as pl
from jax.experimental.pallas import tpu as pltpu
from jax.experimental.pallas import tpu_sc as plsc

B = 16384
D = 32
NC = 2
NS = 16
NW = NC * NS
BPW = B // NW
PASS = 16
NPASS = BPW // PASS


def _sc_body(user_hbm, item_hbm, uf_hbm, if_hbm, out_hbm,
             uidx, iidx, ubufs_a, ibufs_a, ubufs_b, ibufs_b,
             ubufs_c, ibufs_c, sbuf, out_v,
             sem_ua, sem_ia, sem_ub, sem_ib, sem_uc, sem_ic):
    wid = lax.axis_index("s") * NC + lax.axis_index("c")
    base = wid * BPW

    pltpu.sync_copy(user_hbm.at[pl.ds(base, BPW)], uidx)
    pltpu.sync_copy(item_hbm.at[pl.ds(base, BPW)], iidx)

    lane = lax.broadcasted_iota(jnp.int32, (16,), 0)

    def fire(j0, ubufs, ibufs, sem_u, sem_i):
        uvec = uidx[pl.ds(j0, 16)]
        ivec = iidx[pl.ds(j0, 16)]
        for t in range(16):
            ur0 = pl.multiple_of((uvec[t] >> 3) << 3, 8)
            ir0 = pl.multiple_of((ivec[t] >> 3) << 3, 8)
            pltpu.async_copy(uf_hbm.at[pl.ds(ur0, 8)],
                             ubufs.at[pl.ds(t * 8, 8)], sem_u)
            pltpu.async_copy(if_hbm.at[pl.ds(ir0, 8)],
                             ibufs.at[pl.ds(t * 8, 8)], sem_i)

    def drain(ubufs, ibufs, sem_u, sem_i):
        pltpu.make_async_copy(uf_hbm.at[pl.ds(0, PASS * 8)], ubufs,
                              sem_u).wait()
        pltpu.make_async_copy(if_hbm.at[pl.ds(0, PASS * 8)], ibufs,
                              sem_i).wait()

    def fold(j0, ubufs, ibufs):
        uvec = uidx[pl.ds(j0, 16)]
        ivec = iidx[pl.ds(j0, 16)]
        for t in range(16):
            ru = t * 8 + (uvec[t] & 7)
            ri = t * 8 + (ivec[t] & 7)
            a = ubufs[ru, pl.ds(0, 16)] * ibufs[ri, pl.ds(0, 16)]
            bb = ubufs[ru, pl.ds(16, 16)] * ibufs[ri, pl.ds(16, 16)]
            sbuf[pl.ds((j0 + t) * 16, 16)] = a + bb

    A = (ubufs_a, ibufs_a, sem_ua, sem_ia)
    Bb = (ubufs_b, ibufs_b, sem_ub, sem_ib)
    C = (ubufs_c, ibufs_c, sem_uc, sem_ic)

    def step(j_fire, fire_set, j_fold, fold_set):
        if j_fire is not None:
            fire(j_fire, *fire_set)
        drain(fold_set[0], fold_set[1], fold_set[2], fold_set[3])
        fold(j_fold, fold_set[0], fold_set[1])

    fire(0, *A)
    fire(PASS, *Bb)

    def triple(q, _):
        j = q * (3 * PASS)
        step(j + 2 * PASS, C, j, A)
        step(j + 3 * PASS, A, j + PASS, Bb)
        step(j + 4 * PASS, Bb, j + 2 * PASS, C)
        return 0

    nq = (NPASS - 2) // 3
    lax.fori_loop(0, nq, triple, 0)
    step(None, None, nq * 3 * PASS, A)
    step(None, None, nq * 3 * PASS + PASS, Bb)

    def group(g, _):
        base16 = g * 256 + lane * 16
        acc = plsc.load_gather(sbuf, [base16])
        for j in range(1, 16):
            acc = acc + plsc.load_gather(sbuf, [base16 + j])
        out_v[pl.ds(g * 16, 16)] = acc
        return 0

    lax.fori_loop(0, BPW // 16, group, 0)

    pltpu.sync_copy(out_v, out_hbm.at[pl.ds(base, BPW)])


@jax.jit
def kernel(user, item, user_factors, item_factors):
    mesh = plsc.VectorSubcoreMesh(core_axis_name="c", subcore_axis_name="s")
    run = pl.kernel(
        _sc_body,
        out_type=jax.ShapeDtypeStruct((B,), jnp.float32),
        mesh=mesh,
        scratch_types=[
            pltpu.VMEM((BPW,), jnp.int32),
            pltpu.VMEM((BPW,), jnp.int32),
            pltpu.VMEM((PASS * 8, D), jnp.float32),
            pltpu.VMEM((PASS * 8, D), jnp.float32),
            pltpu.VMEM((PASS * 8, D), jnp.float32),
            pltpu.VMEM((PASS * 8, D), jnp.float32),
            pltpu.VMEM((PASS * 8, D), jnp.float32),
            pltpu.VMEM((PASS * 8, D), jnp.float32),
            pltpu.VMEM((BPW * 16,), jnp.float32),
            pltpu.VMEM((BPW,), jnp.float32),
            pltpu.SemaphoreType.DMA,
            pltpu.SemaphoreType.DMA,
            pltpu.SemaphoreType.DMA,
            pltpu.SemaphoreType.DMA,
            pltpu.SemaphoreType.DMA,
            pltpu.SemaphoreType.DMA,
        ],
        compiler_params=pltpu.CompilerParams(needs_layout_passes=False),
    )
    return run(user.astype(jnp.int32), item.astype(jnp.int32),
               user_factors, item_factors)
